# Initial kernel scaffold; baseline (speedup 1.0000x reference)
import jax
import jax.numpy as jnp
from jax.experimental import pallas as pl


def kernel(x, centroids):
    raise NotImplementedError("write your pallas kernel here")


# speedup vs baseline: 1.6485x; 1.6485x over previous
import functools

import jax
import jax.numpy as jnp
from jax import lax
from jax.experimental import pallas as pl
from jax.experimental.pallas import tpu as pltpu
from jax.experimental.pallas import tpu_sc as plsc

N = 16384
D = 256
K = 1024
NB = 16
BN = N // NB

NQ = 4
QP = N // NQ
CG = 8
CW = D // CG
CH = 2048


def _assign_body(c_ref, x_ref, a_ref, cnt_ref):
    i = pl.program_id(0)
    c = c_ref[...]
    xb = x_ref[...]
    c2 = jnp.sum(c * c, axis=1, keepdims=True)
    x2 = jnp.sum(xb * xb, axis=1)[None, :]
    cx = lax.dot_general(c, xb, (((1,), (1,)), ((), ())),
                         preferred_element_type=jnp.float32)
    d2 = c2 + x2 - 2.0 * cx
    d = jnp.sqrt(jnp.maximum(d2, 0.0))
    ds = d * d
    a = jnp.argmin(ds, axis=0).astype(jnp.int32)
    a_ref[0, 0, :] = a
    ks = lax.broadcasted_iota(jnp.int32, (K, BN), 0)
    cnt = jnp.sum((ks == a[None, :]).astype(jnp.float32), axis=1,
                  keepdims=True)
    cntb = jnp.broadcast_to(cnt, (K, 128))

    @pl.when(i == 0)
    def _():
        cnt_ref[...] = cntb

    @pl.when(i != 0)
    def _():
        cnt_ref[...] += cntb


def _assign(c, x):
    return pl.pallas_call(
        _assign_body,
        grid=(NB,),
        in_specs=[
            pl.BlockSpec((K, D), lambda i: (0, 0)),
            pl.BlockSpec((BN, D), lambda i: (i, 0)),
        ],
        out_specs=[
            pl.BlockSpec((1, 1, BN), lambda i: (i, 0, 0)),
            pl.BlockSpec((K, 128), lambda i: (0, 0)),
        ],
        out_shape=[
            jax.ShapeDtypeStruct((NB, 1, BN), jnp.int32),
            jax.ShapeDtypeStruct((K, 128), jnp.float32),
        ],
    )(c, x)


def _segsum_body(x_hbm, a_hbm, out_hbm, tab_v, a_v, x_v):
    ci = lax.axis_index("c")
    si = lax.axis_index("s")
    q = ci * 2 + si // 8
    g = si % 8
    col0 = g * CW
    pt0 = q * QP

    zeros16 = jnp.zeros((16,), jnp.float32)

    def zb(r, _):
        tab_v[r, pl.ds(0, 16)] = zeros16
        tab_v[r, pl.ds(16, 16)] = zeros16
        return 0

    lax.fori_loop(0, K, zb, 0)

    pltpu.sync_copy(a_hbm, a_v)

    def chunk(chi, _):
        pt = pt0 + chi * CH
        arow = pt // CH
        pltpu.sync_copy(x_hbm.at[pl.ds(pt, CH), pl.ds(col0, CW)], x_v)

        def pts(jo, _):
            base = jo * 16
            av16 = a_v[arow, pl.ds(base, 16)]
            for u in range(16):
                row = av16[u]
                xlo = x_v[base + u, pl.ds(0, 16)]
                xhi = x_v[base + u, pl.ds(16, 16)]
                plsc.addupdate(tab_v.at[row, pl.ds(0, 16)], xlo)
                plsc.addupdate(tab_v.at[row, pl.ds(16, 16)], xhi)
            return 0

        lax.fori_loop(0, CH // 16, pts, 0)
        return 0

    lax.fori_loop(0, QP // CH, chunk, 0)
    pltpu.sync_copy(tab_v, out_hbm.at[q, :, pl.ds(col0, CW)])


def _segsum(x, assignment_rows):
    mesh = plsc.VectorSubcoreMesh(core_axis_name="c", subcore_axis_name="s",
                                  num_cores=2, num_subcores=16)
    f = functools.partial(
        pl.kernel,
        out_type=jax.ShapeDtypeStruct((NQ, K, D), jnp.float32),
        mesh=mesh,
        compiler_params=pltpu.CompilerParams(use_tc_tiling_on_sc=False),
        scratch_types=[
            pltpu.VMEM((K, CW), jnp.float32),
            pltpu.VMEM((N // CH, CH), jnp.int32),
            pltpu.VMEM((CH, CW), jnp.float32),
        ],
    )(_segsum_body)
    return f(x, assignment_rows)


def _combine_body(p_ref, cnt_ref, out_ref):
    s = p_ref[0] + p_ref[1] + p_ref[2] + p_ref[3]
    out_ref[...] = s / cnt_ref[:, 0:1]


def _combine(partials, counts):
    return pl.pallas_call(
        _combine_body,
        out_shape=jax.ShapeDtypeStruct((K, D), jnp.float32),
    )(partials, counts)


def kernel(x, centroids):
    c = centroids.reshape(K, D)
    a3, counts = _assign(c, x)
    assignment = a3.reshape(N)
    partials = _segsum(x, assignment.reshape(N // CH, CH))
    means = _combine(partials, counts)
    return assignment, means.reshape(K, 1, D)


# speedup vs baseline: 2.2980x; 1.3940x over previous
import functools

import jax
import jax.numpy as jnp
from jax import lax
from jax.experimental import pallas as pl
from jax.experimental.pallas import tpu as pltpu
from jax.experimental.pallas import tpu_sc as plsc

N = 16384
D = 256
K = 1024
BN = 1024
NBH = 8
NH = N // 2

CW = 16
HQ = NH // 2
CH = 1024
NCH = HQ // CH


def _assign_body(c_ref, x_ref, a_ref, cnt_ref):
    i = pl.program_id(0)
    c = c_ref[...]
    xb = x_ref[...]
    c2 = jnp.sum(c * c, axis=1, keepdims=True)
    x2 = jnp.sum(xb * xb, axis=1)[None, :]
    cx = lax.dot_general(c, xb, (((1,), (1,)), ((), ())),
                         preferred_element_type=jnp.float32)
    d2 = c2 + x2 - 2.0 * cx
    a = jnp.argmin(d2, axis=0).astype(jnp.int32)
    a_ref[0, 0, :] = a
    ks = lax.broadcasted_iota(jnp.int32, (K, BN), 0)
    cnt = jnp.sum((ks == a[None, :]).astype(jnp.float32), axis=1,
                  keepdims=True)
    cntb = jnp.broadcast_to(cnt, (K, 128))

    @pl.when(i == 0)
    def _():
        cnt_ref[...] = cntb

    @pl.when(i != 0)
    def _():
        cnt_ref[...] += cntb


def _assign(c, x, half):
    off = half * NBH
    return pl.pallas_call(
        _assign_body,
        grid=(NBH,),
        in_specs=[
            pl.BlockSpec((K, D), lambda i: (0, 0)),
            pl.BlockSpec((BN, D), lambda i: (i + off, 0)),
        ],
        out_specs=[
            pl.BlockSpec((1, 1, BN), lambda i: (i, 0, 0)),
            pl.BlockSpec((K, 128), lambda i: (0, 0)),
        ],
        out_shape=[
            jax.ShapeDtypeStruct((NBH, 1, BN), jnp.int32),
            jax.ShapeDtypeStruct((K, 128), jnp.float32),
        ],
    )(c, x)


def _make_segsum_body(poff):
    def body(x_hbm, a_hbm, out_hbm, tab_v, a_v, xa_v, xb_v, sa, sb):
        ci = lax.axis_index("c")
        si = lax.axis_index("s")
        col0 = si * CW
        pt0 = poff + ci * HQ

        zeros16 = jnp.zeros((16,), jnp.float32)

        @plsc.parallel_loop(0, K, unroll=8)
        def _(r):
            tab_v[r, pl.ds(0, 16)] = zeros16

        pltpu.sync_copy(a_hbm, a_v)

        bufs = [xa_v, xb_v]
        sems = [sa, sb]
        descs = [None, None]
        descs[0] = pltpu.async_copy(
            x_hbm.at[pl.ds(pt0, CH), pl.ds(col0, CW)], xa_v, sa)
        for chi in range(NCH):
            b = chi % 2
            if chi + 1 < NCH:
                nb = (chi + 1) % 2
                descs[nb] = pltpu.async_copy(
                    x_hbm.at[pl.ds(pt0 + (chi + 1) * CH, CH),
                             pl.ds(col0, CW)],
                    bufs[nb], sems[nb])
            descs[b].wait()
            cur = bufs[b]
            arow = ci * NCH + chi

            @plsc.parallel_loop(0, CH // 16, unroll=4)
            def _(jo, cur=cur, arow=arow):
                base = jo * 16
                av16 = a_v[arow, pl.ds(base, 16)]
                for u in range(16):
                    row = av16[u]
                    xv = cur[base + u, pl.ds(0, 16)]
                    plsc.addupdate(tab_v.at[row, pl.ds(0, 16)], xv)

        pltpu.sync_copy(tab_v, out_hbm.at[ci, :, pl.ds(col0, CW)])

    return body


def _segsum(x, assignment_rows, half):
    mesh = plsc.VectorSubcoreMesh(core_axis_name="c", subcore_axis_name="s",
                                  num_cores=2, num_subcores=16)
    f = functools.partial(
        pl.kernel,
        out_type=jax.ShapeDtypeStruct((2, K, D), jnp.float32),
        mesh=mesh,
        compiler_params=pltpu.CompilerParams(use_tc_tiling_on_sc=False),
        scratch_types=[
            pltpu.VMEM((K, CW), jnp.float32),
            pltpu.VMEM((NH // CH, CH), jnp.int32),
            pltpu.VMEM((CH, CW), jnp.float32),
            pltpu.VMEM((CH, CW), jnp.float32),
            pltpu.SemaphoreType.DMA,
            pltpu.SemaphoreType.DMA,
        ],
    )(_make_segsum_body(half * NH))
    return f(x, assignment_rows)


def _combine_body(p0_ref, p1_ref, c0_ref, c1_ref, out_ref):
    s = p0_ref[0] + p0_ref[1] + p1_ref[0] + p1_ref[1]
    cnt = c0_ref[:, 0:1] + c1_ref[:, 0:1]
    out_ref[...] = s / cnt


def _combine(p0, p1, c0, c1):
    return pl.pallas_call(
        _combine_body,
        out_shape=jax.ShapeDtypeStruct((K, D), jnp.float32),
    )(p0, p1, c0, c1)


def kernel(x, centroids):
    c = centroids.reshape(K, D)
    a3_0, cnt0 = _assign(c, x, 0)
    a3_1, cnt1 = _assign(c, x, 1)
    p0 = _segsum(x, a3_0.reshape(NH // CH, CH), 0)
    p1 = _segsum(x, a3_1.reshape(NH // CH, CH), 1)
    means = _combine(p0, p1, cnt0, cnt1)
    assignment = jnp.concatenate([a3_0.reshape(NH), a3_1.reshape(NH)])
    return assignment, means.reshape(K, 1, D)
